# baseline (speedup 1.0000x reference)
import functools

import jax
import jax.numpy as jnp
from jax import lax
from jax.experimental import pallas as pl
from jax.experimental.pallas import tpu as pltpu
from jax.experimental.pallas import tpu_sc as plsc

V = 1000
VP = 1024
N_TOKENS = 4096 * 20
NC, NS, L = 2, 16, 16
NW = NC * NS
CHUNK = N_TOKENS // NW
GATHER = 128
N_GATHERS = CHUNK // GATHER
RB = 8


def _t2_body(tab_ref, x_ref, y_ref, out_ref, av_ref):
    t = tab_ref[...]
    m = jnp.max(t, axis=1, keepdims=True)
    s = jnp.sum(jnp.exp(t - m), axis=1, keepdims=True)
    t2 = t - (jnp.log(s) + m)
    t2p = jnp.concatenate([t2, jnp.zeros((V, VP - V), jnp.float32)], axis=1)
    out_ref[...] = t2p.reshape(V * VP // 128, 128)
    av = x_ref[...] * VP + y_ref[...]
    av_ref[...] = av.T.reshape(N_TOKENS // 128, 128)


def _sc_body(avf, t2f, out, av, tv, accv, sht, sem_t, sem_s):
    wid = lax.axis_index("c") * NS + lax.axis_index("s")
    sid = lax.axis_index("s")
    base = wid * CHUNK

    QTR = V * VP // 4

    @pl.when(sid < 4)
    def _():
        q = pl.ds(sid * QTR, QTR)
        pltpu.make_async_copy(t2f.at[q], sht.at[q], sem_s).start()

    pltpu.sync_copy(avf.at[pl.ds(base, CHUNK)], av)

    @pl.when(sid < 4)
    def _():
        q = pl.ds(sid * QTR, QTR)
        pltpu.make_async_copy(t2f.at[q], sht.at[q], sem_s).wait()

    plsc.subcore_barrier()

    def fire(c, carry):
        sl = pl.ds(c * GATHER, GATHER)
        pltpu.make_async_copy(sht.at[av.at[sl]], tv.at[sl], sem_t).start()
        return carry

    lax.fori_loop(0, N_GATHERS, fire, 0)

    def drain(c, acc):
        sl = pl.ds(c * GATHER, GATHER)
        pltpu.make_async_copy(sht.at[av.at[sl]], tv.at[sl], sem_t).wait()

        def add(i, a):
            return a + tv[pl.ds(c * GATHER + i * L, L)]

        return lax.fori_loop(0, GATHER // L, add, acc)

    acc = lax.fori_loop(0, N_GATHERS, drain, jnp.zeros((L,), jnp.float32))
    accv[...] = acc
    pltpu.sync_copy(accv, out.at[wid])


def _final_body(part_ref, out_ref):
    out_ref[...] = jnp.sum(part_ref[...], keepdims=True) * (-1.0 / N_TOKENS)


def kernel(X, Y, table):
    t2p, avp = pl.pallas_call(
        _t2_body,
        out_shape=(
            jax.ShapeDtypeStruct((V * VP // 128, 128), jnp.float32),
            jax.ShapeDtypeStruct((N_TOKENS // 128, 128), jnp.int32),
        ),
    )(table, X, Y)
    t2f = t2p.reshape(-1)
    avf = avp.reshape(-1)

    mesh = plsc.VectorSubcoreMesh(
        core_axis_name="c", subcore_axis_name="s", num_cores=NC, num_subcores=NS
    )
    partials = pl.kernel(
        _sc_body,
        out_type=jax.ShapeDtypeStruct((NW, L), jnp.float32),
        mesh=mesh,
        scratch_types=[
            pltpu.VMEM((CHUNK,), jnp.int32),
            pltpu.VMEM((CHUNK,), jnp.float32),
            pltpu.VMEM((L,), jnp.float32),
            pltpu.VMEM_SHARED((V * VP,), jnp.float32),
            pltpu.SemaphoreType.DMA,
            pltpu.SemaphoreType.DMA,
        ],
    )(avf, t2f)

    loss = pl.pallas_call(
        _final_body,
        out_shape=jax.ShapeDtypeStruct((1, 1), jnp.float32),
    )(partials)
    return loss[0, 0]


# speedup vs baseline: 1.0669x; 1.0669x over previous
import functools

import jax
import jax.numpy as jnp
from jax import lax
from jax.experimental import pallas as pl
from jax.experimental.pallas import tpu as pltpu
from jax.experimental.pallas import tpu_sc as plsc

V = 1000
VP = 1024
N_TOKENS = 4096 * 20
NC, NS, L = 2, 16, 16
NW = NC * NS
CHUNK = N_TOKENS // NW
GATHER = 128
N_GATHERS = CHUNK // GATHER
RB = 8


def _t2_body(tab_ref, out_ref):
    t = tab_ref[...]
    m = jnp.max(t, axis=1, keepdims=True)
    s = jnp.sum(jnp.exp(t - m), axis=1, keepdims=True)
    t2 = t - (jnp.log(s) + m)
    t2p = jnp.concatenate([t2, jnp.zeros((V, VP - V), jnp.float32)], axis=1)
    out_ref[...] = t2p.reshape(V * VP // 128, 128)


def _sc_body(avf, t2f, out, av, tv, accv, sht, sem_t, sem_s):
    wid = lax.axis_index("c") * NS + lax.axis_index("s")
    sid = lax.axis_index("s")
    base = wid * CHUNK

    QTR = V * VP // 4

    @pl.when(sid < 4)
    def _():
        q = pl.ds(sid * QTR, QTR)
        pltpu.make_async_copy(t2f.at[q], sht.at[q], sem_s).start()

    pltpu.sync_copy(avf.at[pl.ds(base, CHUNK)], av)

    @pl.when(sid < 4)
    def _():
        q = pl.ds(sid * QTR, QTR)
        pltpu.make_async_copy(t2f.at[q], sht.at[q], sem_s).wait()

    plsc.subcore_barrier()

    def fire(c, carry):
        sl = pl.ds(c * GATHER, GATHER)
        pltpu.make_async_copy(sht.at[av.at[sl]], tv.at[sl], sem_t).start()
        return carry

    lax.fori_loop(0, N_GATHERS, fire, 0)

    def drain(c, acc):
        sl = pl.ds(c * GATHER, GATHER)
        pltpu.make_async_copy(sht.at[av.at[sl]], tv.at[sl], sem_t).wait()

        def add(i, a):
            return a + tv[pl.ds(c * GATHER + i * L, L)]

        return lax.fori_loop(0, GATHER // L, add, acc)

    acc = lax.fori_loop(0, N_GATHERS, drain, jnp.zeros((L,), jnp.float32))
    accv[...] = acc
    pltpu.sync_copy(accv, out.at[wid])


def _final_body(part_ref, out_ref):
    out_ref[...] = jnp.sum(part_ref[...], keepdims=True) * (-1.0 / N_TOKENS)


def kernel(X, Y, table):
    avf = (X * VP + Y).reshape(-1)

    t2p = pl.pallas_call(
        _t2_body,
        out_shape=jax.ShapeDtypeStruct((V * VP // 128, 128), jnp.float32),
    )(table)
    t2f = t2p.reshape(-1)

    mesh = plsc.VectorSubcoreMesh(
        core_axis_name="c", subcore_axis_name="s", num_cores=NC, num_subcores=NS
    )
    partials = pl.kernel(
        _sc_body,
        out_type=jax.ShapeDtypeStruct((NW, L), jnp.float32),
        mesh=mesh,
        scratch_types=[
            pltpu.VMEM((CHUNK,), jnp.int32),
            pltpu.VMEM((CHUNK,), jnp.float32),
            pltpu.VMEM((L,), jnp.float32),
            pltpu.VMEM_SHARED((V * VP,), jnp.float32),
            pltpu.SemaphoreType.DMA,
            pltpu.SemaphoreType.DMA,
        ],
    )(avf, t2f)

    loss = pl.pallas_call(
        _final_body,
        out_shape=jax.ShapeDtypeStruct((1, 1), jnp.float32),
    )(partials)
    return loss[0, 0]
